# baseline (speedup 1.0000x reference)
import functools

import jax
import jax.numpy as jnp
from jax import lax
from jax.experimental import pallas as pl
from jax.experimental.pallas import tpu as pltpu
from jax.experimental.pallas import tpu_sc as plsc

_TOPK = 5
_B = 128
_N = 100000
_CHUNK = 10000
_NCHUNK = _N // _CHUNK
_VECS = _CHUNK // 16
_NC = 2
_NS = 16
_NW = _NC * _NS
_RPW = _B // _NW


def _sc_count_kernel(x_hbm, y_hbm, out_hbm, yall_v, idx_v, y_v, v_v, buf,
                     out_v, sem):
  wid = lax.axis_index("s") * _NC + lax.axis_index("c")
  base = wid * _RPW
  iota = lax.iota(jnp.int32, 16)

  pltpu.sync_copy(y_hbm, yall_v)
  rows = base + jnp.minimum(iota, _RPW - 1)
  ytake = plsc.load_gather(yall_v, [rows])
  y_v[...] = ytake
  idx_v[...] = rows * _N + ytake
  pltpu.async_copy(x_hbm.at[idx_v], v_v, sem).wait()

  yreg = y_v[...]
  vreg = v_v[...]
  total = jnp.int32(0)
  for r in range(_RPW):
    s_vec = jnp.full((16,), yreg[r], jnp.int32)
    v_vec = jnp.full((16,), vreg[r], jnp.float32)
    row_off = (base + r) * _N
    rank = jnp.int32(0)
    for c in range(_NCHUNK):
      pltpu.sync_copy(x_hbm.at[pl.ds(row_off + c * _CHUNK, _CHUNK)], buf)

      def body(k, carry, s_vec=s_vec, v_vec=v_vec, cbase=c * _CHUNK):
        acc, col = carry
        xv = buf[pl.ds(k * 16, 16)]
        beat = jnp.where(col < s_vec, xv >= v_vec, xv > v_vec)
        return acc + beat.astype(jnp.int32), col + 16

      acc, _ = lax.fori_loop(
          0, _VECS, body,
          (jnp.zeros((16,), jnp.int32), c * _CHUNK + iota))
      rank = rank + jnp.sum(acc)
    total = total + (rank < _TOPK).astype(jnp.int32)

  out_v[...] = (iota == 0).astype(jnp.float32) * total.astype(jnp.float32)
  pltpu.sync_copy(out_v, out_hbm.at[wid])


@functools.partial(
    pl.kernel,
    out_type=jax.ShapeDtypeStruct((_NW, 16), jnp.float32),
    mesh=plsc.VectorSubcoreMesh(core_axis_name="c", subcore_axis_name="s",
                                num_cores=_NC, num_subcores=_NS),
    scratch_types=[
        pltpu.VMEM((_B,), jnp.int32),
        pltpu.VMEM((16,), jnp.int32),
        pltpu.VMEM((16,), jnp.int32),
        pltpu.VMEM((16,), jnp.float32),
        pltpu.VMEM((_CHUNK,), jnp.float32),
        pltpu.VMEM((16,), jnp.float32),
        pltpu.SemaphoreType.DMA,
    ],
    compiler_params=pltpu.CompilerParams(needs_layout_passes=False),
)
def _fused_acc(x_hbm, y_hbm, out_hbm, yall_v, idx_v, y_v, v_v, buf, out_v,
               sem):
  _sc_count_kernel(x_hbm, y_hbm, out_hbm, yall_v, idx_v, y_v, v_v, buf,
                   out_v, sem)


def kernel(x, y):
  partial = _fused_acc(x.reshape(-1), y)
  return jnp.sum(partial) / x.shape[0]


# speedup vs baseline: 1.4190x; 1.4190x over previous
import functools

import jax
import jax.numpy as jnp
from jax import lax
from jax.experimental import pallas as pl
from jax.experimental.pallas import tpu as pltpu
from jax.experimental.pallas import tpu_sc as plsc

_TOPK = 5
_B = 128
_N = 100000
_CHUNK = 10000
_NCHUNK = _N // _CHUNK
_VECS = _CHUNK // 16
_NC = 2
_NS = 16
_NW = _NC * _NS
_RPW = _B // _NW


def _chunk_counts(bufb, cbase, s, s_vec, v_vec, acc):
  def ge_f():
    def body(k, a):
      xv = bufb[pl.ds(k * 16, 16)]
      return a + (xv >= v_vec).astype(jnp.int32)

    return lax.fori_loop(0, _VECS, body, acc, unroll=25)

  def gt_f():
    def body(k, a):
      xv = bufb[pl.ds(k * 16, 16)]
      return a + (xv > v_vec).astype(jnp.int32)

    return lax.fori_loop(0, _VECS, body, acc, unroll=25)

  def mixed_f():
    iota = lax.iota(jnp.int32, 16)

    def body(k, carry):
      a, col = carry
      xv = bufb[pl.ds(k * 16, 16)]
      beat = jnp.where(col < s_vec, xv >= v_vec, xv > v_vec)
      return a + beat.astype(jnp.int32), col + 16

    a, _ = lax.fori_loop(0, _VECS, body, (acc, cbase + iota), unroll=5)
    return a

  return lax.cond(
      cbase + _CHUNK <= s, ge_f,
      lambda: lax.cond(cbase > s, gt_f, mixed_f))


def _sc_count_kernel(x_hbm, y_hbm, out_hbm, yall_v, idx_v, y_v, v_v, buf0,
                     buf1, out_v, sem0, sem1):
  wid = lax.axis_index("s") * _NC + lax.axis_index("c")
  base = wid * _RPW
  iota = lax.iota(jnp.int32, 16)

  pltpu.sync_copy(y_hbm, yall_v)
  rows = base + jnp.minimum(iota, _RPW - 1)
  ytake = plsc.load_gather(yall_v, [rows])
  y_v[...] = ytake
  idx_v[...] = rows * _N + ytake
  pltpu.async_copy(x_hbm.at[idx_v], v_v, sem0).wait()

  yreg = y_v[...]
  vreg = v_v[...]
  total = jnp.int32(0)
  for r in range(_RPW):
    s = yreg[r]
    s_vec = jnp.full((16,), s, jnp.int32)
    v_vec = jnp.full((16,), vreg[r], jnp.float32)
    row_off = (base + r) * _N

    def start(c, bufb, semb, row_off=row_off):
      pltpu.make_async_copy(
          x_hbm.at[pl.ds(row_off + c * _CHUNK, _CHUNK)], bufb, semb).start()

    def wait(c, bufb, semb, row_off=row_off):
      pltpu.make_async_copy(
          x_hbm.at[pl.ds(row_off + c * _CHUNK, _CHUNK)], bufb, semb).wait()

    start(0, buf0, sem0)

    def pair_body(c0, acc, s=s, s_vec=s_vec, v_vec=v_vec,
                  start=start, wait=wait):
      c = 2 * c0
      wait(c, buf0, sem0)
      start(c + 1, buf1, sem1)
      acc = _chunk_counts(buf0, c * _CHUNK, s, s_vec, v_vec, acc)
      wait(c + 1, buf1, sem1)

      @pl.when(c0 < _NCHUNK // 2 - 1)
      def _():
        start(c + 2, buf0, sem0)

      acc = _chunk_counts(buf1, (c + 1) * _CHUNK, s, s_vec, v_vec, acc)
      return acc

    acc = lax.fori_loop(0, _NCHUNK // 2, pair_body,
                        jnp.zeros((16,), jnp.int32))
    rank = jnp.sum(acc)
    total = total + (rank < _TOPK).astype(jnp.int32)

  out_v[...] = (iota == 0).astype(jnp.float32) * total.astype(jnp.float32)
  pltpu.sync_copy(out_v, out_hbm.at[wid])


@functools.partial(
    pl.kernel,
    out_type=jax.ShapeDtypeStruct((_NW, 16), jnp.float32),
    mesh=plsc.VectorSubcoreMesh(core_axis_name="c", subcore_axis_name="s",
                                num_cores=_NC, num_subcores=_NS),
    scratch_types=[
        pltpu.VMEM((_B,), jnp.int32),
        pltpu.VMEM((16,), jnp.int32),
        pltpu.VMEM((16,), jnp.int32),
        pltpu.VMEM((16,), jnp.float32),
        pltpu.VMEM((_CHUNK,), jnp.float32),
        pltpu.VMEM((_CHUNK,), jnp.float32),
        pltpu.VMEM((16,), jnp.float32),
        pltpu.SemaphoreType.DMA,
        pltpu.SemaphoreType.DMA,
    ],
    compiler_params=pltpu.CompilerParams(needs_layout_passes=False),
)
def _fused_acc(x_hbm, y_hbm, out_hbm, yall_v, idx_v, y_v, v_v, buf0, buf1,
               out_v, sem0, sem1):
  _sc_count_kernel(x_hbm, y_hbm, out_hbm, yall_v, idx_v, y_v, v_v, buf0,
                   buf1, out_v, sem0, sem1)


def kernel(x, y):
  partial = _fused_acc(x.reshape(-1), y)
  return jnp.sum(partial) / x.shape[0]


# speedup vs baseline: 3.7093x; 2.6141x over previous
import functools

import jax
import jax.numpy as jnp
from jax import lax
from jax.experimental import pallas as pl
from jax.experimental.pallas import tpu as pltpu
from jax.experimental.pallas import tpu_sc as plsc

_TOPK = 5
_B = 128
_N = 100000
_NC = 2
_NS = 16
_NW = _NC * _NS
_RG = 8
_NG = _B // _RG
_HALF = 50048
_CT = 23 * 128
_NCH = _HALF // _CT
_VECS = _CT // 16


def _row_chunk_count(buf, p, gc0, s, s_vec, v_vec):
  def ge_f():
    def body(k, a):
      xv = buf[p, pl.ds(k * 16, 16)]
      return a + (xv >= v_vec).astype(jnp.int32)

    return lax.fori_loop(0, _VECS, body, jnp.zeros((16,), jnp.int32),
                         unroll=8)

  def gt_f():
    def body(k, a):
      xv = buf[p, pl.ds(k * 16, 16)]
      return a + (xv > v_vec).astype(jnp.int32)

    return lax.fori_loop(0, _VECS, body, jnp.zeros((16,), jnp.int32),
                         unroll=8)

  def mixed_f():
    iota = lax.iota(jnp.int32, 16)

    def body(k, carry):
      a, col = carry
      xv = buf[p, pl.ds(k * 16, 16)]
      beat = jnp.where(col < s_vec, xv >= v_vec, xv > v_vec)
      beat = jnp.logical_and(beat, col < _N)
      return a + beat.astype(jnp.int32), col + 16

    a, _ = lax.fori_loop(
        0, _VECS, body,
        (jnp.zeros((16,), jnp.int32), gc0 + iota), unroll=4)
    return a

  acc = lax.cond(
      gc0 + _CT <= s, ge_f,
      lambda: lax.cond(jnp.logical_and(gc0 > s, gc0 + _CT <= _N),
                       gt_f, mixed_f))
  return jnp.sum(acc)


def _sc_rank_kernel(x_hbm, y_hbm, out_hbm, yall_v, ltile, buf, out_v,
                    semc, semt):
  g = lax.axis_index("s")
  h = lax.axis_index("c")
  iota = lax.iota(jnp.int32, 16)
  row0 = pl.multiple_of(g * _RG, _RG)
  col_base = pl.multiple_of(h * _HALF, 128)

  def chunk_copy(c):
    coff = pl.multiple_of(col_base + c * _CT, 128)
    return pltpu.make_async_copy(
        x_hbm.at[pl.ds(row0, _RG), pl.ds(coff, _CT)], buf, semc)

  chunk_copy(0).start()

  pltpu.sync_copy(y_hbm, yall_v)
  ytake = plsc.load_gather(yall_v, [jnp.minimum(row0 + iota, _B - 1)])

  s_list, tile_col = [], []
  for p in range(_RG):
    s = ytake[p]
    c0 = pl.multiple_of((s // 128) * 128, 128)
    s_list.append(s)
    tile_col.append(c0)
    pltpu.make_async_copy(
        x_hbm.at[pl.ds(row0, _RG), pl.ds(c0, 128)], ltile.at[p], semt
    ).start()
  v_list, sv_list = [], []
  for p in range(_RG):
    pltpu.make_async_copy(
        x_hbm.at[pl.ds(row0, _RG), pl.ds(tile_col[p], 128)], ltile.at[p],
        semt).wait()
    v_vec = plsc.load_gather(
        ltile.at[p],
        [jnp.full((16,), p, jnp.int32),
         jnp.full((16,), s_list[p] - tile_col[p], jnp.int32)])
    v_list.append(v_vec)
    sv_list.append(jnp.full((16,), s_list[p], jnp.int32))

  chunk_copy(0).wait()

  def cond_fn(carry):
    c = carry[0]
    rs = carry[1:]
    unresolved = functools.reduce(
        jnp.logical_or, [r < _TOPK for r in rs])
    return jnp.logical_and(c < _NCH, unresolved)

  def body_fn(carry):
    c = carry[0]
    rs = list(carry[1:])

    @pl.when(c > 0)
    def _():
      pltpu.sync_copy(
          x_hbm.at[pl.ds(row0, _RG),
                   pl.ds(pl.multiple_of(col_base + c * _CT, 128), _CT)],
          buf)

    gc0 = col_base + c * _CT
    for p in range(_RG):
      inc = lax.cond(
          rs[p] < _TOPK,
          lambda p=p: _row_chunk_count(buf, p, gc0, s_list[p], sv_list[p],
                                       v_list[p]),
          lambda: jnp.int32(0))
      rs[p] = rs[p] + inc
    return (c + 1, *rs)

  init = (jnp.int32(0),) + tuple(jnp.int32(0) for _ in range(_RG))
  carry = lax.while_loop(cond_fn, body_fn, init)
  rs = carry[1:]

  outvec = jnp.zeros((16,), jnp.float32)
  for p in range(_RG):
    outvec = outvec + (iota == p).astype(jnp.float32) * rs[p].astype(
        jnp.float32)
  out_v[...] = outvec
  pltpu.sync_copy(out_v, out_hbm.at[g * _NC + h])


@functools.partial(
    pl.kernel,
    out_type=jax.ShapeDtypeStruct((_NW, 16), jnp.float32),
    mesh=plsc.VectorSubcoreMesh(core_axis_name="c", subcore_axis_name="s",
                                num_cores=_NC, num_subcores=_NS),
    scratch_types=[
        pltpu.VMEM((_B,), jnp.int32),
        pltpu.VMEM((_RG, _RG, 128), jnp.float32),
        pltpu.VMEM((_RG, _CT), jnp.float32),
        pltpu.VMEM((16,), jnp.float32),
        pltpu.SemaphoreType.DMA,
        pltpu.SemaphoreType.DMA,
    ],
    compiler_params=pltpu.CompilerParams(needs_layout_passes=False),
)
def _fused_acc(x_hbm, y_hbm, out_hbm, yall_v, ltile, buf, out_v, semc,
               semt):
  _sc_rank_kernel(x_hbm, y_hbm, out_hbm, yall_v, ltile, buf, out_v, semc,
                  semt)


def kernel(x, y):
  partial = _fused_acc(x, y)
  ranks = partial.reshape(_NG, _NC, 16)[:, :, :_RG].sum(axis=1)
  correct = (ranks.reshape(-1) < _TOPK).astype(jnp.float32)
  return jnp.sum(correct) / x.shape[0]


# speedup vs baseline: 5.5001x; 1.4828x over previous
import functools

import jax
import jax.numpy as jnp
from jax import lax
from jax.experimental import pallas as pl
from jax.experimental.pallas import tpu as pltpu
from jax.experimental.pallas import tpu_sc as plsc

_TOPK = 5
_B = 128
_N = 100000
_NC = 2
_NS = 16
_NW = _NC * _NS
_NBAND = _B // 16
_CW = 184
_NCHUNK = _N // _CW
_TAIL = _N - _NCHUNK * _CW
_CPW = 17


def _band_window_count(buf, ncols, c0, b, s_vec, v_vec, smin, smax, acc):
  def ge_f():
    def body(k, a):
      xv = buf[k, pl.ds(b * 16, 16)]
      return a + (xv >= v_vec).astype(jnp.int32)

    return lax.fori_loop(0, ncols, body, acc, unroll=8)

  def gt_f():
    def body(k, a):
      xv = buf[k, pl.ds(b * 16, 16)]
      return a + (xv > v_vec).astype(jnp.int32)

    return lax.fori_loop(0, ncols, body, acc, unroll=8)

  def mixed_f():
    def body(k, carry):
      a, colv = carry
      xv = buf[k, pl.ds(b * 16, 16)]
      beat = jnp.where(colv < s_vec, xv >= v_vec, xv > v_vec)
      return a + beat.astype(jnp.int32), colv + 1

    a, _ = lax.fori_loop(
        0, ncols, body, (acc, jnp.full((16,), c0, jnp.int32)), unroll=4)
    return a

  return lax.cond(
      c0 + ncols <= smin, ge_f,
      lambda: lax.cond(c0 > smax, gt_f, mixed_f))


def _sc_rank_kernel(xt_hbm, y_hbm, out_hbm, yall_v, vtab, buf, out_v,
                    semc, semg):
  wid = lax.axis_index("s") * _NC + lax.axis_index("c")
  base_chunk = wid * _CPW
  nchunks = jnp.where(wid == _NW - 1, _CPW - 1, _CPW)

  def chunk_copy(c):
    col0 = pl.multiple_of((base_chunk + c) * _CW, 8)
    return pltpu.make_async_copy(
        xt_hbm.at[pl.ds(col0, _CW), :], buf, semc)

  chunk_copy(0).start()

  pltpu.sync_copy(y_hbm, yall_v)
  pltpu.async_copy(xt_hbm.at[yall_v], vtab, semg).wait()
  iota = lax.iota(jnp.int32, 16)
  s_bands, v_bands, smin_b, smax_b = [], [], [], []
  for b in range(_NBAND):
    s_vec = yall_v[pl.ds(b * 16, 16)]
    v_vec = plsc.load_gather(vtab, [b * 16 + iota, b * 16 + iota])
    s_bands.append(s_vec)
    v_bands.append(v_vec)
    smin_b.append(jnp.min(s_vec))
    smax_b.append(jnp.max(s_vec))

  chunk_copy(0).wait()

  def cond_fn(carry):
    c, done = carry[0], carry[1]
    return jnp.logical_and(c < nchunks, jnp.logical_not(done))

  def body_fn(carry):
    c = carry[0]
    accs = list(carry[2:])

    @pl.when(c > 0)
    def _():
      chunk_copy(c).start()
      chunk_copy(c).wait()

    c0 = (base_chunk + c) * _CW
    for b in range(_NBAND):
      accs[b] = _band_window_count(buf, _CW, c0, b, s_bands[b], v_bands[b],
                                   smin_b[b], smax_b[b], accs[b])
    m = accs[0]
    for b in range(1, _NBAND):
      m = jnp.minimum(m, accs[b])
    done = jnp.min(m) >= _TOPK
    return (c + 1, done, *accs)

  init = (jnp.int32(0), jnp.bool_(False)) + tuple(
      jnp.zeros((16,), jnp.int32) for _ in range(_NBAND))
  carry = lax.while_loop(cond_fn, body_fn, init)
  done = carry[1]
  accs = list(carry[2:])

  @pl.when(jnp.logical_and(wid == _NW - 1, jnp.logical_not(done)))
  def _():
    pltpu.sync_copy(xt_hbm.at[pl.ds(_NCHUNK * _CW, _TAIL), :],
                    buf.at[pl.ds(0, _TAIL), :])
    taccs = []
    for b in range(_NBAND):
      def mixed_tail(b=b):
        def body(k, carry):
          a, colv = carry
          xv = buf[k, pl.ds(b * 16, 16)]
          beat = jnp.where(colv < s_bands[b], xv >= v_bands[b],
                           xv > v_bands[b])
          return a + beat.astype(jnp.int32), colv + 1

        a, _ = lax.fori_loop(
            0, _TAIL, body,
            (accs[b], jnp.full((16,), _NCHUNK * _CW, jnp.int32)),
            unroll=4)
        return a

      taccs.append(mixed_tail())
    for b in range(_NBAND):
      out_v[pl.ds(b * 16, 16)] = taccs[b].astype(jnp.float32)

  @pl.when(jnp.logical_not(
      jnp.logical_and(wid == _NW - 1, jnp.logical_not(done))))
  def _():
    for b in range(_NBAND):
      out_v[pl.ds(b * 16, 16)] = accs[b].astype(jnp.float32)

  pltpu.sync_copy(out_v, out_hbm.at[wid])


@functools.partial(
    pl.kernel,
    out_type=jax.ShapeDtypeStruct((_NW, _B), jnp.float32),
    mesh=plsc.VectorSubcoreMesh(core_axis_name="c", subcore_axis_name="s",
                                num_cores=_NC, num_subcores=_NS),
    scratch_types=[
        pltpu.VMEM((_B,), jnp.int32),
        pltpu.VMEM((_B, _B), jnp.float32),
        pltpu.VMEM((_CW, _B), jnp.float32),
        pltpu.VMEM((_B,), jnp.float32),
        pltpu.SemaphoreType.DMA,
        pltpu.SemaphoreType.DMA,
    ],
    compiler_params=pltpu.CompilerParams(needs_layout_passes=False,
                                         use_tc_tiling_on_sc=True),
)
def _fused_acc(xt_hbm, y_hbm, out_hbm, yall_v, vtab, buf, out_v, semc,
               semg):
  _sc_rank_kernel(xt_hbm, y_hbm, out_hbm, yall_v, vtab, buf, out_v,
                  semc, semg)


def kernel(x, y):
  partial = _fused_acc(x.T, y)
  ranks = partial.sum(axis=0)
  return jnp.sum((ranks < _TOPK).astype(jnp.float32)) / x.shape[0]


# speedup vs baseline: 5.7355x; 1.0428x over previous
import functools

import jax
import jax.numpy as jnp
from jax import lax
from jax.experimental import pallas as pl
from jax.experimental.pallas import tpu as pltpu
from jax.experimental.pallas import tpu_sc as plsc

_TOPK = 5
_B = 128
_N = 100000
_NC = 2
_NS = 16
_NW = _NC * _NS
_NBAND = _B // 16
_CW = 184
_NCHUNK = _N // _CW
_TAIL = _N - _NCHUNK * _CW
_CPW = 17


def _band_window_count(buf, ncols, c0, b, s_vec, v_vec, smin, smax, acc):
  def ge_f():
    def body(k, a):
      xv = buf[pl.ds(k * _B + b * 16, 16)]
      return a + (xv >= v_vec).astype(jnp.int32)

    return lax.fori_loop(0, ncols, body, acc, unroll=8)

  def gt_f():
    def body(k, a):
      xv = buf[pl.ds(k * _B + b * 16, 16)]
      return a + (xv > v_vec).astype(jnp.int32)

    return lax.fori_loop(0, ncols, body, acc, unroll=8)

  def mixed_f():
    def body(k, carry):
      a, colv = carry
      xv = buf[pl.ds(k * _B + b * 16, 16)]
      beat = jnp.where(colv < s_vec, xv >= v_vec, xv > v_vec)
      return a + beat.astype(jnp.int32), colv + 1

    a, _ = lax.fori_loop(
        0, ncols, body, (acc, jnp.full((16,), c0, jnp.int32)), unroll=4)
    return a

  return lax.cond(
      c0 + ncols <= smin, ge_f,
      lambda: lax.cond(c0 > smax, gt_f, mixed_f))


def _sc_rank_kernel(xflat_hbm, y_hbm, out_hbm, yall_v, idx_v,
                    vflat_v, buf0, buf1, out_v, semc0, semc1, semg):
  wid = lax.axis_index("s") * _NC + lax.axis_index("c")
  base_chunk = wid * _CPW
  nchunks = jnp.where(wid == _NW - 1, _CPW - 1, _CPW)

  def chunk_copy(c, bufb, semb):
    w0 = pl.multiple_of((base_chunk + c) * _CW * _B, 8)
    return pltpu.make_async_copy(
        xflat_hbm.at[pl.ds(w0, _CW * _B)], bufb, semb)

  chunk_copy(0, buf0, semc0).start()

  pltpu.sync_copy(y_hbm, yall_v)
  iota = lax.iota(jnp.int32, 16)
  s_bands, smin_b, smax_b = [], [], []
  for b in range(_NBAND):
    s_vec = yall_v[pl.ds(b * 16, 16)]
    s_bands.append(s_vec)
    smin_b.append(jnp.min(s_vec))
    smax_b.append(jnp.max(s_vec))
    idx_v[pl.ds(b * 16, 16)] = (
        (s_vec >> 3) * 1024 + (s_vec & 7) * 128 + (b * 16 + iota))
  pltpu.async_copy(xflat_hbm.at[idx_v], vflat_v, semg).wait()
  v_bands = [vflat_v[pl.ds(b * 16, 16)] for b in range(_NBAND)]

  def all_done(accs):
    m = accs[0]
    for b in range(1, _NBAND):
      m = jnp.minimum(m, accs[b])
    return jnp.min(m) >= _TOPK

  def process(bufb, c, accs):
    c0 = (base_chunk + c) * _CW
    return [
        _band_window_count(bufb, _CW, c0, b, s_bands[b], v_bands[b],
                           smin_b[b], smax_b[b], accs[b])
        for b in range(_NBAND)
    ]

  def cond_fn(carry):
    c, done = carry[0], carry[1]
    return jnp.logical_and(c < nchunks, jnp.logical_not(done))

  def body_fn(carry):
    c = carry[0]
    accs = list(carry[2:])

    chunk_copy(c, buf0, semc0).wait()

    @pl.when(c + 1 < nchunks)
    def _():
      chunk_copy(c + 1, buf1, semc1).start()

    accs = process(buf0, c, accs)
    skip1 = jnp.logical_or(all_done(accs), c + 1 >= nchunks)

    @pl.when(c + 1 < nchunks)
    def _():
      chunk_copy(c + 1, buf1, semc1).wait()

    accs = lax.cond(skip1, lambda a: list(a),
                    lambda a: process(buf1, c + 1, a), accs)
    resolved = all_done(accs)

    @pl.when(jnp.logical_and(c + 2 < nchunks, jnp.logical_not(resolved)))
    def _():
      chunk_copy(c + 2, buf0, semc0).start()

    return (c + 2, resolved, *accs)

  init = (jnp.int32(0), jnp.bool_(False)) + tuple(
      jnp.zeros((16,), jnp.int32) for _ in range(_NBAND))
  carry = lax.while_loop(cond_fn, body_fn, init)
  done = carry[1]
  accs = list(carry[2:])

  @pl.when(jnp.logical_and(wid == _NW - 1, jnp.logical_not(done)))
  def _():
    pltpu.sync_copy(xflat_hbm.at[pl.ds(_NCHUNK * _CW * _B, _TAIL * _B)],
                    buf0.at[pl.ds(0, _TAIL * _B)])
    taccs = []
    for b in range(_NBAND):
      def mixed_tail(b=b):
        def body(k, carry):
          a, colv = carry
          xv = buf0[pl.ds(k * _B + b * 16, 16)]
          beat = jnp.where(colv < s_bands[b], xv >= v_bands[b],
                           xv > v_bands[b])
          return a + beat.astype(jnp.int32), colv + 1

        a, _ = lax.fori_loop(
            0, _TAIL, body,
            (accs[b], jnp.full((16,), _NCHUNK * _CW, jnp.int32)),
            unroll=4)
        return a

      taccs.append(mixed_tail())
    for b in range(_NBAND):
      out_v[pl.ds(b * 16, 16)] = taccs[b].astype(jnp.float32)

  @pl.when(jnp.logical_not(
      jnp.logical_and(wid == _NW - 1, jnp.logical_not(done))))
  def _():
    for b in range(_NBAND):
      out_v[pl.ds(b * 16, 16)] = accs[b].astype(jnp.float32)

  pltpu.sync_copy(out_v, out_hbm.at[wid])


@functools.partial(
    pl.kernel,
    out_type=jax.ShapeDtypeStruct((_NW, _B), jnp.float32),
    mesh=plsc.VectorSubcoreMesh(core_axis_name="c", subcore_axis_name="s",
                                num_cores=_NC, num_subcores=_NS),
    scratch_types=[
        pltpu.VMEM((_B,), jnp.int32),
        pltpu.VMEM((_B,), jnp.int32),
        pltpu.VMEM((_B,), jnp.float32),
        pltpu.VMEM((_CW * _B,), jnp.float32),
        pltpu.VMEM((_CW * _B,), jnp.float32),
        pltpu.VMEM((_B,), jnp.float32),
        pltpu.SemaphoreType.DMA,
        pltpu.SemaphoreType.DMA,
        pltpu.SemaphoreType.DMA,
    ],
    compiler_params=pltpu.CompilerParams(needs_layout_passes=False,
                                         use_tc_tiling_on_sc=True),
)
def _fused_acc(xflat_hbm, y_hbm, out_hbm, yall_v, idx_v, vflat_v,
               buf0, buf1, out_v, semc0, semc1, semg):
  _sc_rank_kernel(xflat_hbm, y_hbm, out_hbm, yall_v, idx_v,
                  vflat_v, buf0, buf1, out_v, semc0, semc1, semg)


def kernel(x, y):
  partial = _fused_acc(x.T.reshape(-1), y)
  ranks = partial.sum(axis=0)
  return jnp.sum((ranks < _TOPK).astype(jnp.float32)) / x.shape[0]


# speedup vs baseline: 6.7305x; 1.1735x over previous
import functools

import jax
import jax.numpy as jnp
from jax import lax
from jax.experimental import pallas as pl
from jax.experimental.pallas import tpu as pltpu
from jax.experimental.pallas import tpu_sc as plsc

_TOPK = 5
_B = 128
_N = 100000
_NC = 2
_NS = 16
_NW = _NC * _NS
_NBAND = _B // 16
_CW = 184
_NCHUNK = _N // _CW
_TAIL = _N - _NCHUNK * _CW
_CPW = 17


def _band_window_count(buf, ncols, c0, b, s_vec, v_vec, smin, smax, acc):
  def ge_f():
    def body(k, a):
      xv = buf[pl.ds(k * _B + b * 16, 16)]
      return a + (xv >= v_vec).astype(jnp.int32)

    return lax.fori_loop(0, ncols, body, acc, unroll=8)

  def gt_f():
    def body(k, a):
      xv = buf[pl.ds(k * _B + b * 16, 16)]
      return a + (xv > v_vec).astype(jnp.int32)

    return lax.fori_loop(0, ncols, body, acc, unroll=8)

  def mixed_f():
    def body(k, carry):
      a, colv = carry
      xv = buf[pl.ds(k * _B + b * 16, 16)]
      beat = jnp.where(colv < s_vec, xv >= v_vec, xv > v_vec)
      return a + beat.astype(jnp.int32), colv + 1

    a, _ = lax.fori_loop(
        0, ncols, body, (acc, jnp.full((16,), c0, jnp.int32)), unroll=4)
    return a

  return lax.cond(
      c0 + ncols <= smin, ge_f,
      lambda: lax.cond(c0 > smax, gt_f, mixed_f))


def _sc_rank_kernel(xflat_hbm, y_hbm, out_hbm, yall_v, idx_v,
                    vflat_v, buf0, buf1, out_v, semc0, semc1, semg):
  wid = lax.axis_index("s") * _NC + lax.axis_index("c")
  base_chunk = wid * _CPW
  nchunks = jnp.where(wid == _NW - 1, _CPW - 1, _CPW)

  def chunk_copy(c, bufb, semb):
    w0 = pl.multiple_of((base_chunk + c) * _CW * _B, 8)
    return pltpu.make_async_copy(
        xflat_hbm.at[pl.ds(w0, _CW * _B)], bufb, semb)

  chunk_copy(0, buf0, semc0).start()

  pltpu.sync_copy(y_hbm, yall_v)
  iota = lax.iota(jnp.int32, 16)
  s_bands, smin_b, smax_b = [], [], []
  for b in range(_NBAND):
    s_vec = yall_v[pl.ds(b * 16, 16)]
    s_bands.append(s_vec)
    smin_b.append(jnp.min(s_vec))
    smax_b.append(jnp.max(s_vec))
    idx_v[pl.ds(b * 16, 16)] = (
        (s_vec >> 3) * 1024 + (s_vec & 7) * 128 + (b * 16 + iota))
  pltpu.async_copy(xflat_hbm.at[idx_v], vflat_v, semg).wait()
  v_bands = [vflat_v[pl.ds(b * 16, 16)] for b in range(_NBAND)]

  def all_done(accs):
    m = accs[0]
    for b in range(1, _NBAND):
      m = jnp.minimum(m, accs[b])
    return jnp.min(m) >= _TOPK

  def process(bufb, c, accs):
    c0 = (base_chunk + c) * _CW
    out = []
    for b in range(_NBAND):
      out.append(lax.cond(
          jnp.min(accs[b]) >= _TOPK,
          lambda a: a,
          lambda a, b=b, c0=c0: _band_window_count(
              bufb, _CW, c0, b, s_bands[b], v_bands[b],
              smin_b[b], smax_b[b], a),
          accs[b]))
    return out

  def cond_fn(carry):
    c, done = carry[0], carry[1]
    return jnp.logical_and(c < nchunks, jnp.logical_not(done))

  def body_fn(carry):
    c = carry[0]
    accs = list(carry[2:])

    chunk_copy(c, buf0, semc0).wait()

    @pl.when(c + 1 < nchunks)
    def _():
      chunk_copy(c + 1, buf1, semc1).start()

    accs = process(buf0, c, accs)
    skip1 = jnp.logical_or(all_done(accs), c + 1 >= nchunks)

    @pl.when(c + 1 < nchunks)
    def _():
      chunk_copy(c + 1, buf1, semc1).wait()

    accs = lax.cond(skip1, lambda a: list(a),
                    lambda a: process(buf1, c + 1, a), accs)
    resolved = all_done(accs)

    @pl.when(jnp.logical_and(c + 2 < nchunks, jnp.logical_not(resolved)))
    def _():
      chunk_copy(c + 2, buf0, semc0).start()

    return (c + 2, resolved, *accs)

  init = (jnp.int32(0), jnp.bool_(False)) + tuple(
      jnp.zeros((16,), jnp.int32) for _ in range(_NBAND))
  carry = lax.while_loop(cond_fn, body_fn, init)
  done = carry[1]
  accs = list(carry[2:])

  @pl.when(jnp.logical_and(wid == _NW - 1, jnp.logical_not(done)))
  def _():
    pltpu.sync_copy(xflat_hbm.at[pl.ds(_NCHUNK * _CW * _B, _TAIL * _B)],
                    buf0.at[pl.ds(0, _TAIL * _B)])
    taccs = []
    for b in range(_NBAND):
      def mixed_tail(b=b):
        def body(k, carry):
          a, colv = carry
          xv = buf0[pl.ds(k * _B + b * 16, 16)]
          beat = jnp.where(colv < s_bands[b], xv >= v_bands[b],
                           xv > v_bands[b])
          return a + beat.astype(jnp.int32), colv + 1

        a, _ = lax.fori_loop(
            0, _TAIL, body,
            (accs[b], jnp.full((16,), _NCHUNK * _CW, jnp.int32)),
            unroll=4)
        return a

      taccs.append(mixed_tail())
    for b in range(_NBAND):
      out_v[pl.ds(b * 16, 16)] = taccs[b].astype(jnp.float32)

  @pl.when(jnp.logical_not(
      jnp.logical_and(wid == _NW - 1, jnp.logical_not(done))))
  def _():
    for b in range(_NBAND):
      out_v[pl.ds(b * 16, 16)] = accs[b].astype(jnp.float32)

  pltpu.sync_copy(out_v, out_hbm.at[wid])


@functools.partial(
    pl.kernel,
    out_type=jax.ShapeDtypeStruct((_NW, _B), jnp.float32),
    mesh=plsc.VectorSubcoreMesh(core_axis_name="c", subcore_axis_name="s",
                                num_cores=_NC, num_subcores=_NS),
    scratch_types=[
        pltpu.VMEM((_B,), jnp.int32),
        pltpu.VMEM((_B,), jnp.int32),
        pltpu.VMEM((_B,), jnp.float32),
        pltpu.VMEM((_CW * _B,), jnp.float32),
        pltpu.VMEM((_CW * _B,), jnp.float32),
        pltpu.VMEM((_B,), jnp.float32),
        pltpu.SemaphoreType.DMA,
        pltpu.SemaphoreType.DMA,
        pltpu.SemaphoreType.DMA,
    ],
    compiler_params=pltpu.CompilerParams(needs_layout_passes=False,
                                         use_tc_tiling_on_sc=True),
)
def _fused_acc(xflat_hbm, y_hbm, out_hbm, yall_v, idx_v, vflat_v,
               buf0, buf1, out_v, semc0, semc1, semg):
  _sc_rank_kernel(xflat_hbm, y_hbm, out_hbm, yall_v, idx_v,
                  vflat_v, buf0, buf1, out_v, semc0, semc1, semg)


def kernel(x, y):
  partial = _fused_acc(x.T.reshape(-1), y)
  ranks = partial.sum(axis=0)
  return jnp.sum((ranks < _TOPK).astype(jnp.float32)) / x.shape[0]


# speedup vs baseline: 6.8156x; 1.0127x over previous
import functools

import jax
import jax.numpy as jnp
from jax import lax
from jax.experimental import pallas as pl
from jax.experimental.pallas import tpu as pltpu
from jax.experimental.pallas import tpu_sc as plsc

_TOPK = 5
_B = 128
_N = 100000
_NC = 2
_NS = 16
_NW = _NC * _NS
_NBAND = _B // 16
_CW = 184
_NCHUNK = _N // _CW
_TAIL = _N - _NCHUNK * _CW
_CPW = 17


def _band_window_count(buf, ncols, c0, b, s_vec, v_vec, smin, smax, acc):
  def ge_f():
    def body(k, a):
      xv = buf[pl.ds(k * _B + b * 16, 16)]
      return a + (xv >= v_vec).astype(jnp.int32)

    return lax.fori_loop(0, ncols, body, acc, unroll=8)

  def gt_f():
    def body(k, a):
      xv = buf[pl.ds(k * _B + b * 16, 16)]
      return a + (xv > v_vec).astype(jnp.int32)

    return lax.fori_loop(0, ncols, body, acc, unroll=8)

  def mixed_f():
    def body(k, carry):
      a, colv = carry
      xv = buf[pl.ds(k * _B + b * 16, 16)]
      beat = jnp.where(colv < s_vec, xv >= v_vec, xv > v_vec)
      return a + beat.astype(jnp.int32), colv + 1

    a, _ = lax.fori_loop(
        0, ncols, body, (acc, jnp.full((16,), c0, jnp.int32)), unroll=4)
    return a

  return lax.cond(
      c0 + ncols <= smin, ge_f,
      lambda: lax.cond(c0 > smax, gt_f, mixed_f))


def _sc_rank_kernel(xflat_hbm, y_hbm, out_hbm, yall_v, idx_v,
                    vflat_v, buf0, buf1, out_v, semc0, semc1, semg):
  wid = lax.axis_index("s") * _NC + lax.axis_index("c")
  base_chunk = wid * _CPW
  nchunks = jnp.where(wid == _NW - 1, _CPW - 1, _CPW)

  def chunk_copy(c, bufb, semb):
    w0 = pl.multiple_of((base_chunk + c) * _CW * _B, 8)
    return pltpu.make_async_copy(
        xflat_hbm.at[pl.ds(w0, _CW * _B)], bufb, semb)

  chunk_copy(0, buf0, semc0).start()

  pltpu.sync_copy(y_hbm, yall_v)
  iota = lax.iota(jnp.int32, 16)
  s_bands, smin_b, smax_b = [], [], []
  for b in range(_NBAND):
    s_vec = yall_v[pl.ds(b * 16, 16)]
    s_bands.append(s_vec)
    smin_b.append(jnp.min(s_vec))
    smax_b.append(jnp.max(s_vec))
    idx_v[pl.ds(b * 16, 16)] = (
        (s_vec >> 3) * 1024 + (s_vec & 7) * 128 + (b * 16 + iota))
  pltpu.async_copy(xflat_hbm.at[idx_v], vflat_v, semg).wait()
  v_bands = [vflat_v[pl.ds(b * 16, 16)] for b in range(_NBAND)]

  def all_done(accs):
    m = accs[0]
    for b in range(1, _NBAND):
      m = jnp.minimum(m, accs[b])
    return jnp.min(m) >= _TOPK

  def process(bufb, c, accs):
    c0 = (base_chunk + c) * _CW
    out = []
    for b in range(_NBAND):
      out.append(lax.cond(
          jnp.min(accs[b]) >= _TOPK,
          lambda a: a,
          lambda a, b=b, c0=c0: _band_window_count(
              bufb, _CW, c0, b, s_bands[b], v_bands[b],
              smin_b[b], smax_b[b], a),
          accs[b]))
    return out

  def cond_fn(carry):
    c, done = carry[0], carry[1]
    return jnp.logical_and(c < nchunks, jnp.logical_not(done))

  def body_fn(carry):
    c = carry[0]
    accs = list(carry[2:])

    chunk_copy(c, buf0, semc0).wait()

    @pl.when(c + 1 < nchunks)
    def _():
      chunk_copy(c + 1, buf1, semc1).start()

    accs = process(buf0, c, accs)
    skip1 = jnp.logical_or(all_done(accs), c + 1 >= nchunks)

    @pl.when(c + 1 < nchunks)
    def _():
      chunk_copy(c + 1, buf1, semc1).wait()

    @pl.when(c + 2 < nchunks)
    def _():
      chunk_copy(c + 2, buf0, semc0).start()

    accs = lax.cond(skip1, lambda a: list(a),
                    lambda a: process(buf1, c + 1, a), accs)
    resolved = all_done(accs)

    return (c + 2, resolved, *accs)

  init = (jnp.int32(0), jnp.bool_(False)) + tuple(
      jnp.zeros((16,), jnp.int32) for _ in range(_NBAND))
  carry = lax.while_loop(cond_fn, body_fn, init)
  done = carry[1]
  accs = list(carry[2:])

  @pl.when(carry[0] < nchunks)
  def _():
    chunk_copy(carry[0], buf0, semc0).wait()

  @pl.when(jnp.logical_and(wid == _NW - 1, jnp.logical_not(done)))
  def _():
    pltpu.sync_copy(xflat_hbm.at[pl.ds(_NCHUNK * _CW * _B, _TAIL * _B)],
                    buf0.at[pl.ds(0, _TAIL * _B)])
    taccs = []
    for b in range(_NBAND):
      def mixed_tail(b=b):
        def body(k, carry):
          a, colv = carry
          xv = buf0[pl.ds(k * _B + b * 16, 16)]
          beat = jnp.where(colv < s_bands[b], xv >= v_bands[b],
                           xv > v_bands[b])
          return a + beat.astype(jnp.int32), colv + 1

        a, _ = lax.fori_loop(
            0, _TAIL, body,
            (accs[b], jnp.full((16,), _NCHUNK * _CW, jnp.int32)),
            unroll=4)
        return a

      taccs.append(mixed_tail())
    for b in range(_NBAND):
      out_v[pl.ds(b * 16, 16)] = taccs[b].astype(jnp.float32)

  @pl.when(jnp.logical_not(
      jnp.logical_and(wid == _NW - 1, jnp.logical_not(done))))
  def _():
    for b in range(_NBAND):
      out_v[pl.ds(b * 16, 16)] = accs[b].astype(jnp.float32)

  pltpu.sync_copy(out_v, out_hbm.at[wid])


@functools.partial(
    pl.kernel,
    out_type=jax.ShapeDtypeStruct((_NW, _B), jnp.float32),
    mesh=plsc.VectorSubcoreMesh(core_axis_name="c", subcore_axis_name="s",
                                num_cores=_NC, num_subcores=_NS),
    scratch_types=[
        pltpu.VMEM((_B,), jnp.int32),
        pltpu.VMEM((_B,), jnp.int32),
        pltpu.VMEM((_B,), jnp.float32),
        pltpu.VMEM((_CW * _B,), jnp.float32),
        pltpu.VMEM((_CW * _B,), jnp.float32),
        pltpu.VMEM((_B,), jnp.float32),
        pltpu.SemaphoreType.DMA,
        pltpu.SemaphoreType.DMA,
        pltpu.SemaphoreType.DMA,
    ],
    compiler_params=pltpu.CompilerParams(needs_layout_passes=False,
                                         use_tc_tiling_on_sc=True),
)
def _fused_acc(xflat_hbm, y_hbm, out_hbm, yall_v, idx_v, vflat_v,
               buf0, buf1, out_v, semc0, semc1, semg):
  _sc_rank_kernel(xflat_hbm, y_hbm, out_hbm, yall_v, idx_v,
                  vflat_v, buf0, buf1, out_v, semc0, semc1, semg)


def kernel(x, y):
  partial = _fused_acc(x.T.reshape(-1), y)
  ranks = partial.sum(axis=0)
  return jnp.sum((ranks < _TOPK).astype(jnp.float32)) / x.shape[0]
